# Initial kernel scaffold; baseline (speedup 1.0000x reference)
import jax
import jax.numpy as jnp
from jax.experimental import pallas as pl


def kernel(x, weight):
    raise NotImplementedError("write your pallas kernel here")


# speedup vs baseline: 4.1357x; 4.1357x over previous
import functools

import jax
import jax.numpy as jnp
from jax import lax
from jax.experimental import pallas as pl
from jax.experimental.pallas import tpu as pltpu
from jax.experimental.pallas import tpu_sc as plsc

_VOCAB = 100000
_D = 64
_B = 4096 * 200
_NC = 2
_NS = 16
_NW = _NC * _NS
_PER_W = _B // _NW
_SUB = 128
_K = 8
_CH = _SUB * _K
_NCH = _PER_W // _CH
_XROWS_PER_W = _PER_W // _SUB


def _body(x_hbm, w_hbm, out_hbm, idx_v, rows_v, gsem):
    wid = lax.axis_index("s") * _NC + lax.axis_index("c")
    xrow0 = wid * _XROWS_PER_W
    out0 = wid * _PER_W

    @pl.loop(0, _NCH)
    def _chunk(g):
        pltpu.sync_copy(x_hbm.at[pl.ds(xrow0 + g * _K, _K)], idx_v)
        for j in range(_K):
            pltpu.async_copy(
                w_hbm.at[idx_v.at[j]],
                rows_v.at[pl.ds(j * _SUB, _SUB)],
                gsem,
            )
        for j in range(_K):
            pltpu.make_async_copy(
                w_hbm.at[idx_v.at[j]],
                rows_v.at[pl.ds(j * _SUB, _SUB)],
                gsem,
            ).wait()
        pltpu.sync_copy(rows_v, out_hbm.at[pl.ds(out0 + g * _CH, _CH)])


@jax.jit
def _lookup(x2d, weight):
    mesh = plsc.VectorSubcoreMesh(
        core_axis_name="c", subcore_axis_name="s",
        num_cores=_NC, num_subcores=_NS,
    )
    return pl.kernel(
        _body,
        out_type=jax.ShapeDtypeStruct((_B, _D), jnp.float32),
        mesh=mesh,
        scratch_types=[
            pltpu.VMEM((_K, _SUB), jnp.int32),
            pltpu.VMEM((_CH, _D), jnp.float32),
            pltpu.SemaphoreType.DMA,
        ],
        compiler_params=pltpu.CompilerParams(use_tc_tiling_on_sc=False),
    )(x2d, weight)


def kernel(x, weight):
    x2d = x.reshape(_B // _SUB, _SUB)
    out = _lookup(x2d, weight)
    return out.reshape(x.shape[0], x.shape[1], _D)


# speedup vs baseline: 4.2571x; 1.0293x over previous
import jax
import jax.numpy as jnp
from jax import lax
from jax.experimental import pallas as pl
from jax.experimental.pallas import tpu as pltpu
from jax.experimental.pallas import tpu_sc as plsc

_VOCAB = 100000
_D = 64
_B = 4096 * 200
_NC = 2
_NS = 16
_NW = _NC * _NS
_PER_W = _B // _NW
_L = 128
_XR_W = _PER_W // _L
_SL = 4
_NSL = _XR_W // _SL


def _body(x_hbm, w_hbm, out_hbm, idx_v, rows0, rows1, gsem0, gsem1):
    wid = lax.axis_index("s") * _NC + lax.axis_index("c")
    xrow0 = wid * _XR_W

    pltpu.sync_copy(x_hbm.at[pl.ds(xrow0, _XR_W)], idx_v)

    def gather(s, rows, gsem):
        for j in range(_SL):
            pltpu.async_copy(w_hbm.at[idx_v.at[s * _SL + j]], rows.at[j], gsem)

    def drain(s, rows, gsem):
        for j in range(_SL):
            pltpu.make_async_copy(
                w_hbm.at[idx_v.at[s * _SL + j]], rows.at[j], gsem
            ).wait()

    def put(s, rows):
        pltpu.sync_copy(rows, out_hbm.at[pl.ds(xrow0 + s * _SL, _SL)])

    gather(0, rows0, gsem0)
    gather(1, rows1, gsem1)

    @pl.loop(0, _NSL // 2)
    def _slab_pair(t):
        s0 = 2 * t
        drain(s0, rows0, gsem0)
        put(s0, rows0)

        @pl.when(t < _NSL // 2 - 1)
        def _():
            gather(s0 + 2, rows0, gsem0)

        drain(s0 + 1, rows1, gsem1)
        put(s0 + 1, rows1)

        @pl.when(t < _NSL // 2 - 1)
        def _():
            gather(s0 + 3, rows1, gsem1)


@jax.jit
def _lookup(x2d, weight):
    mesh = plsc.VectorSubcoreMesh(
        core_axis_name="c", subcore_axis_name="s",
        num_cores=_NC, num_subcores=_NS,
    )
    return pl.kernel(
        _body,
        out_type=jax.ShapeDtypeStruct((_B // _L, _L, _D), jnp.float32),
        mesh=mesh,
        scratch_types=[
            pltpu.VMEM((_XR_W, _L), jnp.int32),
            pltpu.VMEM((_SL, _L, _D), jnp.float32),
            pltpu.VMEM((_SL, _L, _D), jnp.float32),
            pltpu.SemaphoreType.DMA,
            pltpu.SemaphoreType.DMA,
        ],
        compiler_params=pltpu.CompilerParams(use_tc_tiling_on_sc=False),
    )(x2d, weight)


def kernel(x, weight):
    x2d = x.reshape(_B // _L, _L)
    out = _lookup(x2d, weight)
    return out.reshape(x.shape[0], x.shape[1], _D)
